# baseline (speedup 1.0000x reference)
import functools

import jax
import jax.numpy as jnp
from jax import lax
from jax.experimental import pallas as pl
from jax.experimental.pallas import tpu as pltpu
from jax.experimental.pallas import tpu_sc as plsc

_D = 128
_G = 256
_NW = 32
_NSUB = 16
_CHUNK = 128
_IDXW = 16


@functools.lru_cache(maxsize=None)
def _make_agg(n_pad: int, cpw: int):
    rows_per_tile = n_pad // _NSUB
    mesh = plsc.VectorSubcoreMesh(core_axis_name="c", subcore_axis_name="s")

    assert cpw % _IDXW == 0
    nwin = cpw // _IDXW

    @functools.partial(
        pl.kernel,
        out_type=jax.ShapeDtypeStruct((2, n_pad, _D), jnp.float32),
        mesh=mesh,
        scratch_types=[
            pltpu.VMEM((_IDXW, _CHUNK), jnp.int32),
            pltpu.VMEM((_IDXW, _CHUNK), jnp.int32),
            pltpu.VMEM((_CHUNK, _D), jnp.float32),
            pltpu.VMEM((_CHUNK, _D), jnp.float32),
            pltpu.VMEM_SHARED((n_pad, _D), jnp.float32),
            pltpu.SemaphoreType.DMA,
        ],
    )
    def agg_kernel(h_hbm, src_hbm, dst_hbm, zero_hbm, out_hbm,
                   src_v, dst_v, buf0, buf1, agg_sh, gsem):
        c = lax.axis_index("c")
        s = lax.axis_index("s")
        wid = s * 2 + c

        pltpu.sync_copy(zero_hbm, agg_sh.at[pl.ds(s * rows_per_tile, rows_per_tile)])
        plsc.subcore_barrier()

        def window(w, carry):
            base = w * _IDXW
            pltpu.sync_copy(src_hbm.at[wid, pl.ds(base, _IDXW)], src_v)
            pltpu.sync_copy(dst_hbm.at[wid, pl.ds(base, _IDXW)], dst_v)

            pltpu.async_copy(h_hbm.at[src_v.at[0]], buf0, gsem)

            def body(jj, carry2):
                j0 = jj * 2
                pltpu.make_async_copy(h_hbm.at[src_v.at[j0]], buf0, gsem).wait()
                pltpu.async_copy(h_hbm.at[src_v.at[j0 + 1]], buf1, gsem)
                pltpu.sync_copy(buf0, agg_sh.at[dst_v.at[j0]], add=True)
                pltpu.make_async_copy(h_hbm.at[src_v.at[j0 + 1]], buf1, gsem).wait()

                @pl.when(j0 + 2 < _IDXW)
                def _():
                    pltpu.async_copy(h_hbm.at[src_v.at[j0 + 2]], buf0, gsem)

                pltpu.sync_copy(buf1, agg_sh.at[dst_v.at[j0 + 1]], add=True)
                return carry2

            lax.fori_loop(0, _IDXW // 2, body, 0)
            return carry

        lax.fori_loop(0, nwin, window, 0)
        plsc.subcore_barrier()
        pltpu.sync_copy(agg_sh.at[pl.ds(s * rows_per_tile, rows_per_tile)],
                        out_hbm.at[c, pl.ds(s * rows_per_tile, rows_per_tile)])

    return agg_kernel


def _mlp_layer(h, agg2, w1, b1, w2, b2, block_rows):
    n = h.shape[0]
    nblk = n // block_rows

    def body(h_ref, a0_ref, a1_ref, w1_ref, b1_ref, w2_ref, b2_ref, o_ref):
        hb = h_ref[...]
        z = hb + a0_ref[0] + a1_ref[0]
        t = jnp.dot(z, w1_ref[...], preferred_element_type=jnp.float32) + b1_ref[...]
        t = jnp.maximum(t, 0.0)
        o_ref[...] = hb + jnp.dot(t, w2_ref[...],
                                  preferred_element_type=jnp.float32) + b2_ref[...]

    return pl.pallas_call(
        body,
        grid=(nblk,),
        in_specs=[
            pl.BlockSpec((block_rows, _D), lambda i: (i, 0)),
            pl.BlockSpec((1, block_rows, _D), lambda i: (0, i, 0)),
            pl.BlockSpec((1, block_rows, _D), lambda i: (1, i, 0)),
            pl.BlockSpec((_D, _D), lambda i: (0, 0)),
            pl.BlockSpec((_D,), lambda i: (0,)),
            pl.BlockSpec((_D, _D), lambda i: (0, 0)),
            pl.BlockSpec((_D,), lambda i: (0,)),
        ],
        out_specs=pl.BlockSpec((block_rows, _D), lambda i: (i, 0)),
        out_shape=jax.ShapeDtypeStruct((n, _D), jnp.float32),
    )(h, agg2, agg2, w1, b1, w2, b2)


def _pool_head(h, batch, wf1, bf1, wf2, bf2, block_rows):
    n = h.shape[0]
    nblk = n // block_rows
    batch3 = batch.reshape(nblk, 1, block_rows)
    wf2p = jnp.zeros((_D, 128), jnp.float32).at[:, :1].set(wf2)
    bf2p = jnp.zeros((128,), jnp.float32).at[0].set(bf2[0])

    def body(b_ref, h_ref, wf1_ref, bf1_ref, wf2_ref, bf2_ref, o_ref, sums, cnts):
        i = pl.program_id(0)

        @pl.when(i == 0)
        def _():
            sums[...] = jnp.zeros_like(sums)
            cnts[...] = jnp.zeros_like(cnts)

        b = b_ref[0, 0, :]
        oh_t = (b[None, :] == lax.broadcasted_iota(
            jnp.int32, (_G, block_rows), 0)).astype(jnp.float32)
        sums[...] += jnp.dot(oh_t, h_ref[...], preferred_element_type=jnp.float32)
        cnts[...] += jnp.dot(oh_t, jnp.ones((block_rows, _D), jnp.float32),
                             preferred_element_type=jnp.float32)

        @pl.when(i == nblk - 1)
        def _():
            pooled = sums[...] / jnp.maximum(cnts[...], 1.0)
            f = jnp.dot(pooled, wf1_ref[...],
                        preferred_element_type=jnp.float32) + bf1_ref[...]
            f = jnp.where(f >= 0.0, f, 0.01 * f)
            o_ref[...] = jnp.dot(f, wf2_ref[...],
                                 preferred_element_type=jnp.float32) + bf2_ref[...]

    out = pl.pallas_call(
        body,
        grid=(nblk,),
        in_specs=[
            pl.BlockSpec((1, 1, block_rows), lambda i: (i, 0, 0)),
            pl.BlockSpec((block_rows, _D), lambda i: (i, 0)),
            pl.BlockSpec((_D, _D), lambda i: (0, 0)),
            pl.BlockSpec((_D,), lambda i: (0,)),
            pl.BlockSpec((_D, 128), lambda i: (0, 0)),
            pl.BlockSpec((128,), lambda i: (0,)),
        ],
        out_specs=pl.BlockSpec((_G, 128), lambda i: (0, 0)),
        out_shape=jax.ShapeDtypeStruct((_G, 128), jnp.float32),
        scratch_shapes=[
            pltpu.VMEM((_G, _D), jnp.float32),
            pltpu.VMEM((_G, _D), jnp.float32),
        ],
    )(batch3, h, wf1, bf1, wf2p, bf2p)
    return out[:, :1]


def kernel(x, edge_index, batch,
           W1_0, b1_0, W2_0, b2_0,
           W1_1, b1_1, W2_1, b2_1,
           W1_2, b1_2, W2_2, b2_2,
           Wf1, bf1, Wf2, bf2):
    n, d = x.shape
    e = edge_index.shape[1]
    assert d == _D

    n_pad = (n // 128 + 1) * 128
    cpw = -(-e // (_NW * _CHUNK))
    cpw = -(-cpw // _IDXW) * _IDXW
    e_pad = _NW * cpw * _CHUNK

    e_pad = _NW * cpw * _CHUNK
    trash = n_pad - n
    pad_dst = n + jnp.arange(e_pad - e, dtype=jnp.int32) % trash
    src = jnp.concatenate([edge_index[0], jnp.zeros((e_pad - e,), jnp.int32)])
    dst = jnp.concatenate([edge_index[1], pad_dst])
    srcp = src.reshape(_NW, cpw, _CHUNK)
    dstp = dst.reshape(_NW, cpw, _CHUNK)
    zrows = jnp.zeros((n_pad // _NSUB, _D), jnp.float32)

    agg_fn = _make_agg(n_pad, cpw)
    params = [(W1_0, b1_0, W2_0, b2_0),
              (W1_1, b1_1, W2_1, b2_1),
              (W1_2, b1_2, W2_2, b2_2)]

    h = x
    for (w1, b1, w2, b2) in params:
        agg2 = agg_fn(h, srcp, dstp, zrows)
        h = _mlp_layer(h, agg2, w1, b1, w2, b2, block_rows=1000)

    return _pool_head(h, batch, Wf1, bf1, Wf2, bf2, block_rows=1000)


# speedup vs baseline: 2.8262x; 2.8262x over previous
import functools

import jax
import jax.numpy as jnp
from jax import lax
from jax.experimental import pallas as pl
from jax.experimental.pallas import tpu as pltpu
from jax.experimental.pallas import tpu_sc as plsc

_D = 128
_G = 256
_NW = 32
_NSUB = 16
_CHUNK = 128
_IDXW = 16


@functools.lru_cache(maxsize=None)
def _make_agg(n_pad: int, cpw: int):
    rows_per_tile = n_pad // _NSUB
    mesh = plsc.VectorSubcoreMesh(core_axis_name="c", subcore_axis_name="s")

    assert cpw % _IDXW == 0
    nwin = cpw // _IDXW

    @functools.partial(
        pl.kernel,
        out_type=jax.ShapeDtypeStruct((2, n_pad, _D), jnp.float32),
        mesh=mesh,
        scratch_types=[
            pltpu.VMEM((_IDXW, _CHUNK), jnp.int32),
            pltpu.VMEM((_IDXW, _CHUNK), jnp.int32),
            pltpu.VMEM((_CHUNK, _D), jnp.float32),
            pltpu.VMEM((_CHUNK, _D), jnp.float32),
            pltpu.VMEM_SHARED((n_pad, _D), jnp.float32),
            pltpu.SemaphoreType.DMA,
        ],
    )
    def agg_kernel(h_hbm, src_hbm, dst_hbm, zero_hbm, out_hbm,
                   src_v, dst_v, buf0, buf1, agg_sh, gsem):
        c = lax.axis_index("c")
        s = lax.axis_index("s")
        wid = s * 2 + c

        pltpu.sync_copy(zero_hbm, agg_sh.at[pl.ds(s * rows_per_tile, rows_per_tile)])
        plsc.subcore_barrier()

        def window(w, carry):
            base = w * _IDXW
            pltpu.sync_copy(src_hbm.at[wid, pl.ds(base, _IDXW)], src_v)
            pltpu.sync_copy(dst_hbm.at[wid, pl.ds(base, _IDXW)], dst_v)

            pltpu.async_copy(h_hbm.at[src_v.at[0]], buf0, gsem)

            def body(jj, carry2):
                j0 = jj * 2
                pltpu.make_async_copy(h_hbm.at[src_v.at[j0]], buf0, gsem).wait()
                pltpu.async_copy(h_hbm.at[src_v.at[j0 + 1]], buf1, gsem)
                pltpu.sync_copy(buf0, agg_sh.at[dst_v.at[j0]], add=True)
                pltpu.make_async_copy(h_hbm.at[src_v.at[j0 + 1]], buf1, gsem).wait()

                @pl.when(j0 + 2 < _IDXW)
                def _():
                    pltpu.async_copy(h_hbm.at[src_v.at[j0 + 2]], buf0, gsem)

                pltpu.sync_copy(buf1, agg_sh.at[dst_v.at[j0 + 1]], add=True)
                return carry2

            lax.fori_loop(0, _IDXW // 2, body, 0)
            return carry

        lax.fori_loop(0, nwin, window, 0)
        plsc.subcore_barrier()
        pltpu.sync_copy(agg_sh.at[pl.ds(s * rows_per_tile, rows_per_tile)],
                        out_hbm.at[c, pl.ds(s * rows_per_tile, rows_per_tile)])

    return agg_kernel


def _mlp_layer(h, agg2, w1, b1, w2, b2, block_rows):
    n = h.shape[0]
    nblk = n // block_rows

    def body(h_ref, a0_ref, a1_ref, w1_ref, b1_ref, w2_ref, b2_ref, o_ref):
        hb = h_ref[...]
        z = hb + a0_ref[0] + a1_ref[0]
        t = jnp.dot(z, w1_ref[...], preferred_element_type=jnp.float32) + b1_ref[...]
        t = jnp.maximum(t, 0.0)
        o_ref[...] = hb + jnp.dot(t, w2_ref[...],
                                  preferred_element_type=jnp.float32) + b2_ref[...]

    return pl.pallas_call(
        body,
        grid=(nblk,),
        in_specs=[
            pl.BlockSpec((block_rows, _D), lambda i: (i, 0)),
            pl.BlockSpec((1, block_rows, _D), lambda i: (0, i, 0)),
            pl.BlockSpec((1, block_rows, _D), lambda i: (1, i, 0)),
            pl.BlockSpec((_D, _D), lambda i: (0, 0)),
            pl.BlockSpec((_D,), lambda i: (0,)),
            pl.BlockSpec((_D, _D), lambda i: (0, 0)),
            pl.BlockSpec((_D,), lambda i: (0,)),
        ],
        out_specs=pl.BlockSpec((block_rows, _D), lambda i: (i, 0)),
        out_shape=jax.ShapeDtypeStruct((n, _D), jnp.float32),
    )(h, agg2, agg2, w1, b1, w2, b2)


def _pool_head(h, batch, wf1, bf1, wf2, bf2, block_rows):
    n = h.shape[0]
    nblk = n // block_rows
    batch3 = batch.reshape(nblk, 1, block_rows)
    wf2p = jnp.zeros((_D, 128), jnp.float32).at[:, :1].set(wf2)
    bf2p = jnp.zeros((128,), jnp.float32).at[0].set(bf2[0])

    def body(b_ref, h_ref, wf1_ref, bf1_ref, wf2_ref, bf2_ref, o_ref, sums, cnts):
        i = pl.program_id(0)

        @pl.when(i == 0)
        def _():
            sums[...] = jnp.zeros_like(sums)
            cnts[...] = jnp.zeros_like(cnts)

        b = b_ref[0, 0, :]
        oh_t = (b[None, :] == lax.broadcasted_iota(
            jnp.int32, (_G, block_rows), 0)).astype(jnp.float32)
        sums[...] += jnp.dot(oh_t, h_ref[...], preferred_element_type=jnp.float32)
        cnts[...] += jnp.dot(oh_t, jnp.ones((block_rows, _D), jnp.float32),
                             preferred_element_type=jnp.float32)

        @pl.when(i == nblk - 1)
        def _():
            pooled = sums[...] / jnp.maximum(cnts[...], 1.0)
            f = jnp.dot(pooled, wf1_ref[...],
                        preferred_element_type=jnp.float32) + bf1_ref[...]
            f = jnp.where(f >= 0.0, f, 0.01 * f)
            o_ref[...] = jnp.dot(f, wf2_ref[...],
                                 preferred_element_type=jnp.float32) + bf2_ref[...]

    out = pl.pallas_call(
        body,
        grid=(nblk,),
        in_specs=[
            pl.BlockSpec((1, 1, block_rows), lambda i: (i, 0, 0)),
            pl.BlockSpec((block_rows, _D), lambda i: (i, 0)),
            pl.BlockSpec((_D, _D), lambda i: (0, 0)),
            pl.BlockSpec((_D,), lambda i: (0,)),
            pl.BlockSpec((_D, 128), lambda i: (0, 0)),
            pl.BlockSpec((128,), lambda i: (0,)),
        ],
        out_specs=pl.BlockSpec((_G, 128), lambda i: (0, 0)),
        out_shape=jax.ShapeDtypeStruct((_G, 128), jnp.float32),
        scratch_shapes=[
            pltpu.VMEM((_G, _D), jnp.float32),
            pltpu.VMEM((_G, _D), jnp.float32),
        ],
    )(batch3, h, wf1, bf1, wf2p, bf2p)
    return out[:, :1]


def kernel(x, edge_index, batch,
           W1_0, b1_0, W2_0, b2_0,
           W1_1, b1_1, W2_1, b2_1,
           W1_2, b1_2, W2_2, b2_2,
           Wf1, bf1, Wf2, bf2):
    n, d = x.shape
    e = edge_index.shape[1]
    assert d == _D

    n_pad = (n // 128 + 1) * 128
    cpw = -(-e // (_NW * _CHUNK))
    cpw = -(-cpw // _IDXW) * _IDXW
    e_pad = _NW * cpw * _CHUNK

    e_pad = _NW * cpw * _CHUNK
    trash = n_pad - n
    pad_dst = n + jnp.arange(e_pad - e, dtype=jnp.int32) % trash
    pad_src = jnp.arange(e_pad - e, dtype=jnp.int32) % n
    src = jnp.concatenate([edge_index[0], pad_src])
    dst = jnp.concatenate([edge_index[1], pad_dst])
    srcp = src.reshape(_NW, cpw, _CHUNK)
    dstp = dst.reshape(_NW, cpw, _CHUNK)
    zrows = jnp.zeros((n_pad // _NSUB, _D), jnp.float32)

    agg_fn = _make_agg(n_pad, cpw)
    params = [(W1_0, b1_0, W2_0, b2_0),
              (W1_1, b1_1, W2_1, b2_1),
              (W1_2, b1_2, W2_2, b2_2)]

    h = x
    for (w1, b1, w2, b2) in params:
        agg2 = agg_fn(h, srcp, dstp, zrows)
        h = _mlp_layer(h, agg2, w1, b1, w2, b2, block_rows=1000)

    return _pool_head(h, batch, Wf1, bf1, Wf2, bf2, block_rows=1000)
